# Initial kernel scaffold; baseline (speedup 1.0000x reference)
import jax
import jax.numpy as jnp
from jax.experimental import pallas as pl


def kernel(nodes, senders, receivers, grid_senders, grid_receivers, n_node, embed, Wg1, bg1, Wg2, bg2, Wm, bm, Wlog, blog, Weval, beval, Wout, bout):
    raise NotImplementedError("write your pallas kernel here")


# speedup vs baseline: 5.6467x; 5.6467x over previous
import functools

import jax
import jax.numpy as jnp
from jax import lax
from jax.experimental import pallas as pl
from jax.experimental.pallas import tpu as pltpu
from jax.experimental.pallas import tpu_sc as plsc

N = 10000
E = 320000
P = 100
INNER = 128
N_GNN = 7
N_EVAL = 5

NC, NS = 2, 16
NF = 10240
ROWS_T = NF // NS
EB = 128
NB = E // EB

_mesh = plsc.VectorSubcoreMesh(core_axis_name="c", subcore_axis_name="s")
f32 = jnp.float32
i32 = jnp.int32


def _nbatches(tid, nworkers):
    rem = NB % nworkers
    return jnp.where(tid < rem, NB // nworkers + 1, NB // nworkers)


_SC_EMBED_KW = dict(
    out_type=jax.ShapeDtypeStruct((NF, INNER), f32),
    mesh=_mesh,
    scratch_types=[
        pltpu.VMEM((EB,), i32),
        pltpu.VMEM((EB, INNER), f32),
        pltpu.SemaphoreType.DMA,
    ],
)


def _sc_embed_body(nodes_h, embed_h, x_o, nidx, xbuf, sem):
    cid = lax.axis_index("c")
    sid = lax.axis_index("s")

    @pl.when(cid == 0)
    def _():
        def gstep(t, c):
            nb2 = sid * ROWS_T + t * EB
            pltpu.sync_copy(nodes_h.at[pl.ds(nb2, EB)], nidx)
            pltpu.async_copy(embed_h.at[nidx], xbuf, sem).wait()
            pltpu.sync_copy(xbuf, x_o.at[pl.ds(nb2, EB)])
            return c

        lax.fori_loop(0, ROWS_T // EB, gstep, 0)


_sc_embed = pl.kernel(_sc_embed_body, **_SC_EMBED_KW)


_SC_CONV_KW = dict(
    out_type=(
        jax.ShapeDtypeStruct((NF, INNER), f32),
        jax.ShapeDtypeStruct((NF, INNER), f32),
    ),
    mesh=_mesh,
    scratch_types=[
        pltpu.VMEM((EB,), i32),
        pltpu.VMEM((EB,), i32),
        pltpu.VMEM((EB, INNER), f32),
        pltpu.VMEM_SHARED((NF, INNER), f32),
        pltpu.SemaphoreType.DMA,
    ],
)


def _sc_conv_body(p1, p2, s1, r1, s2, r2, m1_o, m2_o, idx_s, idx_r, rows, acc, sem):
    cid = lax.axis_index("c")
    sid = lax.axis_index("s")
    rbase = sid * ROWS_T
    nb = _nbatches(sid, NS)

    def run(p_hbm, s_hbm, r_hbm, m_hbm):
        pltpu.sync_copy(p_hbm.at[pl.ds(rbase, ROWS_T)],
                        acc.at[pl.ds(rbase, ROWS_T)])
        plsc.subcore_barrier()

        def step(i, c):
            off = (sid + i * NS) * EB
            pltpu.sync_copy(s_hbm.at[pl.ds(off, EB)], idx_s)
            pltpu.sync_copy(r_hbm.at[pl.ds(off, EB)], idx_r)
            pltpu.async_copy(p_hbm.at[idx_s], rows, sem).wait()
            pltpu.sync_copy(rows, acc.at[idx_r], add=True)
            return c

        lax.fori_loop(0, nb, step, 0)
        plsc.subcore_barrier()
        pltpu.sync_copy(acc.at[pl.ds(rbase, ROWS_T)],
                        m_hbm.at[pl.ds(rbase, ROWS_T)])

    @pl.when(cid == 0)
    def _():
        run(p1, s1, r1, m1_o)

    @pl.when(cid == 1)
    def _():
        run(p2, s2, r2, m2_o)


_sc_conv = pl.kernel(_sc_conv_body, **_SC_CONV_KW)


_SC_EDGE_KW = dict(
    out_type=jax.ShapeDtypeStruct((E,), f32),
    mesh=_mesh,
    scratch_types=[
        pltpu.VMEM((EB,), i32),
        pltpu.VMEM((EB,), i32),
        pltpu.VMEM((EB, INNER), f32),
        pltpu.VMEM((EB, INNER), f32),
        pltpu.VMEM((EB * 16,), f32),
        pltpu.VMEM((EB,), f32),
        pltpu.SemaphoreType.DMA,
    ],
    compiler_params=pltpu.CompilerParams(needs_layout_passes=False),
)


def _sc_edge_body(l_h, s_h, r_h, out_o, idx_s, idx_r, ls, lr, tmp, ob, sem):
    cid = lax.axis_index("c")
    sid = lax.axis_index("s")
    wid = cid * NS + sid
    nb = _nbatches(wid, NC * NS)
    lane = lax.iota(i32, 16)

    def batch(i, c):
        off = (wid + i * NC * NS) * EB
        pltpu.sync_copy(s_h.at[pl.ds(off, EB)], idx_s)
        pltpu.sync_copy(r_h.at[pl.ds(off, EB)], idx_r)
        pltpu.async_copy(l_h.at[idx_s], ls, sem).wait()
        pltpu.async_copy(l_h.at[idx_r], lr, sem).wait()

        def edge(e, c2):
            vs = [ls[e, pl.ds(k * 16, 16)] * lr[e, pl.ds(k * 16, 16)]
                  for k in range(8)]
            while len(vs) > 1:
                vs = [vs[t] + vs[t + 1] for t in range(0, len(vs), 2)]
            tmp[pl.ds(e * 16, 16)] = vs[0]
            return c2

        lax.fori_loop(0, EB, edge, 0)

        def group(g, c2):
            vs = [plsc.load_gather(tmp, [lane * 16 + g * 256 + j])
                  for j in range(16)]
            while len(vs) > 1:
                vs = [vs[t] + vs[t + 1] for t in range(0, len(vs), 2)]
            ob[pl.ds(g * 16, 16)] = vs[0]
            return c2

        lax.fori_loop(0, EB // 16, group, 0)
        pltpu.sync_copy(ob, out_o.at[pl.ds(off, EB)])
        return c

    lax.fori_loop(0, nb, batch, 0)


_sc_edge = pl.kernel(_sc_edge_body, **_SC_EDGE_KW)


def _inv(deg_blk):
    return lax.rsqrt(deg_blk[:, 0:1])


def _tc_pre0_body(x, ds1, ds2, w1, b1, w2, b2, p1_o, p2_o):
    xv = x[...]
    p1_o[...] = (jnp.dot(xv, w1[...], preferred_element_type=f32) + b1[...]) * _inv(ds1[...])
    p2_o[...] = (jnp.dot(xv, w2[...], preferred_element_type=f32) + b2[...]) * _inv(ds2[...])


def _tc_pre0(x, ds1, ds2, w1, b1, w2, b2):
    blk = 1024
    row = pl.BlockSpec((blk, INNER), lambda i: (i, 0))
    deg = pl.BlockSpec((blk, INNER), lambda i: (i, 0))
    ws = pl.BlockSpec((INNER, INNER), lambda i: (0, 0))
    bs = pl.BlockSpec((1, INNER), lambda i: (0, 0))
    return pl.pallas_call(
        _tc_pre0_body,
        grid=(NF // blk,),
        in_specs=[row, deg, deg, ws, bs, ws, bs],
        out_specs=[row, row],
        out_shape=[jax.ShapeDtypeStruct((NF, INNER), f32)] * 2,
    )(x, ds1, ds2, w1, b1, w2, b2)


def _tc_mid_body(m1, m2, dr1, dr2, ds1, ds2, wma, wmb, bm_, w1, b1, w2, b2,
                 p1_o, p2_o):
    h1 = m1[...] * _inv(dr1[...])
    h2 = m2[...] * _inv(dr2[...])
    xv = jnp.dot(h1, wma[...], preferred_element_type=f32)
    xv = xv + jnp.dot(h2, wmb[...], preferred_element_type=f32)
    xv = jnp.maximum(xv + bm_[...], 0.0)
    p1_o[...] = (jnp.dot(xv, w1[...], preferred_element_type=f32) + b1[...]) * _inv(ds1[...])
    p2_o[...] = (jnp.dot(xv, w2[...], preferred_element_type=f32) + b2[...]) * _inv(ds2[...])


def _tc_mid(m1, m2, dr1, dr2, ds1, ds2, wma, wmb, bm_, w1, b1, w2, b2):
    blk = 1024
    row = pl.BlockSpec((blk, INNER), lambda i: (i, 0))
    deg = pl.BlockSpec((blk, INNER), lambda i: (i, 0))
    ws = pl.BlockSpec((INNER, INNER), lambda i: (0, 0))
    bs = pl.BlockSpec((1, INNER), lambda i: (0, 0))
    return pl.pallas_call(
        _tc_mid_body,
        grid=(NF // blk,),
        in_specs=[row, row, deg, deg, deg, deg, ws, ws, bs, ws, bs, ws, bs],
        out_specs=[row, row],
        out_shape=[jax.ShapeDtypeStruct((NF, INNER), f32)] * 2,
    )(m1, m2, dr1, dr2, ds1, ds2, wma, wmb, bm_, w1, b1, w2, b2)


def _tc_fin_body(m1, m2, dr1, dr2, wma, wmb, bm_, wlog, blog_, x_o, l_o):
    h1 = m1[...] * _inv(dr1[...])
    h2 = m2[...] * _inv(dr2[...])
    xv = jnp.dot(h1, wma[...], preferred_element_type=f32)
    xv = xv + jnp.dot(h2, wmb[...], preferred_element_type=f32)
    xv = jnp.maximum(xv + bm_[...], 0.0)
    x_o[...] = xv
    l_o[...] = jnp.dot(xv, wlog[...], preferred_element_type=f32) + blog_[...]


def _tc_fin(m1, m2, dr1, dr2, wma, wmb, bm_, wlog, blog_):
    blk = 1024
    row = pl.BlockSpec((blk, INNER), lambda i: (i, 0))
    deg = pl.BlockSpec((blk, INNER), lambda i: (i, 0))
    ws = pl.BlockSpec((INNER, INNER), lambda i: (0, 0))
    bs = pl.BlockSpec((1, INNER), lambda i: (0, 0))
    return pl.pallas_call(
        _tc_fin_body,
        grid=(NF // blk,),
        in_specs=[row, row, deg, deg, ws, ws, bs, ws, bs],
        out_specs=[row, row],
        out_shape=[jax.ShapeDtypeStruct((NF, INNER), f32)] * 2,
    )(m1, m2, dr1, dr2, wma, wmb, bm_, wlog, blog_)


def _tc_pool_body(x, v_o):
    xb = x[...]
    mask = lax.broadcasted_iota(i32, (1, P, INNER), 1) != 0
    xm = jnp.where(mask, xb, 0.0)
    v_o[...] = jnp.sum(xm, axis=1, keepdims=True)


def _tc_pool(xr):
    return pl.pallas_call(
        _tc_pool_body,
        grid=(P,),
        in_specs=[pl.BlockSpec((1, P, INNER), lambda i: (i, 0, 0))],
        out_specs=pl.BlockSpec((1, 1, INNER), lambda i: (i, 0, 0)),
        out_shape=jax.ShapeDtypeStruct((P, 1, INNER), f32),
    )(xr)


def _tc_head_body(vs, nn, wev, bev, wout, bout_, v_o):
    v = vs[...] * (1.0 / (nn[...] - 1.0))
    wv = wev[...]
    bv = bev[...]
    for l in range(N_EVAL):
        v = jnp.maximum(jnp.dot(v, wv[l], preferred_element_type=f32)
                        + bv[l:l + 1, :], 0.0)
    v_o[...] = jnp.tanh(jnp.dot(v, wout[...], preferred_element_type=f32)
                        + bout_[0, 0])


def _tc_head(vs, nn, wev, bev, wout_pad, bout_):
    full = lambda s: pl.BlockSpec(s, lambda: tuple(0 for _ in s))
    return pl.pallas_call(
        _tc_head_body,
        in_specs=[full((P, INNER)), full((P, 1)), full((N_EVAL, INNER, INNER)),
                  full((N_EVAL, INNER)), full((INNER, INNER)), full((1, 1))],
        out_specs=full((P, INNER)),
        out_shape=jax.ShapeDtypeStruct((P, INNER), f32),
    )(vs, nn, wev, bev, wout_pad, bout_)


def kernel(nodes, senders, receivers, grid_senders, grid_receivers, n_node,
           embed, Wg1, bg1, Wg2, bg2, Wm, bm, Wlog, blog, Weval, beval,
           Wout, bout):
    s1 = senders.astype(i32)
    r1 = receivers.astype(i32)
    s2 = grid_senders.astype(i32)
    r2 = grid_receivers.astype(i32)
    nodes_pad = jnp.zeros((NF,), i32).at[:N].set(nodes.astype(i32))

    x0 = _sc_embed(nodes_pad, embed)
    ones_nf = jnp.ones((NF, INNER), f32)
    dr1, dr2 = _sc_conv(ones_nf, ones_nf, s1, r1, s2, r2)
    ds1, ds2 = _sc_conv(ones_nf, ones_nf, r1, s1, r2, s2)

    p1, p2 = _tc_pre0(x0, ds1, ds2, Wg1[0], bg1[0][None], Wg2[0], bg2[0][None])
    for l in range(N_GNN):
        m1, m2 = _sc_conv(p1, p2, s1, r1, s2, r2)
        wma, wmb = Wm[l, :INNER], Wm[l, INNER:]
        if l < N_GNN - 1:
            p1, p2 = _tc_mid(m1, m2, dr1, dr2, ds1, ds2, wma, wmb,
                             bm[l][None], Wg1[l + 1], bg1[l + 1][None],
                             Wg2[l + 1], bg2[l + 1][None])
        else:
            x7, L = _tc_fin(m1, m2, dr1, dr2, wma, wmb, bm[l][None],
                            Wlog, blog[None])

    logits = _sc_edge(L, s1, r1)
    vsum = _tc_pool(x7[:N].reshape(P, N // P, INNER)).reshape(P, INNER)
    v = _tc_head(vsum, n_node.astype(f32).reshape(P, 1), Weval, beval,
                 jnp.pad(Wout, ((0, 0), (0, INNER - 1))), bout.reshape(1, 1))
    return logits, v[:, :1]


# speedup vs baseline: 9.7937x; 1.7344x over previous
import functools

import jax
import jax.numpy as jnp
from jax import lax
from jax.experimental import pallas as pl
from jax.experimental.pallas import tpu as pltpu
from jax.experimental.pallas import tpu_sc as plsc

N = 10000
E = 320000
P = 100
INNER = 128
N_GNN = 7
N_EVAL = 5

NC, NS = 2, 16
NF = 10240
ROWS_T = NF // NS
EB = 128
NB = E // EB

_mesh = plsc.VectorSubcoreMesh(core_axis_name="c", subcore_axis_name="s")
f32 = jnp.float32
i32 = jnp.int32


def _nbatches(tid, nworkers):
    rem = NB % nworkers
    return jnp.where(tid < rem, NB // nworkers + 1, NB // nworkers)


_SC_EMBED_KW = dict(
    out_type=jax.ShapeDtypeStruct((NF, INNER), f32),
    mesh=_mesh,
    scratch_types=[
        pltpu.VMEM((EB,), i32),
        pltpu.VMEM((EB, INNER), f32),
        pltpu.SemaphoreType.DMA,
    ],
)


def _sc_embed_body(nodes_h, embed_h, x_o, nidx, xbuf, sem):
    cid = lax.axis_index("c")
    sid = lax.axis_index("s")

    @pl.when(cid == 0)
    def _():
        def gstep(t, c):
            nb2 = sid * ROWS_T + t * EB
            pltpu.sync_copy(nodes_h.at[pl.ds(nb2, EB)], nidx)
            pltpu.async_copy(embed_h.at[nidx], xbuf, sem).wait()
            pltpu.sync_copy(xbuf, x_o.at[pl.ds(nb2, EB)])
            return c

        lax.fori_loop(0, ROWS_T // EB, gstep, 0)


_sc_embed = pl.kernel(_sc_embed_body, **_SC_EMBED_KW)


_SC_CONV_KW = dict(
    out_type=(
        jax.ShapeDtypeStruct((NF, INNER), f32),
        jax.ShapeDtypeStruct((NF, INNER), f32),
    ),
    mesh=_mesh,
    scratch_types=[
        pltpu.VMEM((2, EB), i32),
        pltpu.VMEM((2, EB), i32),
        pltpu.VMEM((EB, INNER), f32),
        pltpu.VMEM((EB, INNER), f32),
        pltpu.VMEM_SHARED((NF, INNER), f32),
        pltpu.SemaphoreType.DMA,
        pltpu.SemaphoreType.DMA,
        pltpu.SemaphoreType.DMA,
        pltpu.SemaphoreType.DMA,
        pltpu.SemaphoreType.DMA,
    ],
)


def _sc_conv_body(p1, p2, s1, r1, s2, r2, m1_o, m2_o,
                  idxs, idxr, rows0, rows1, acc,
                  semi, semg0, semg1, sems0, sems1):
    cid = lax.axis_index("c")
    sid = lax.axis_index("s")
    rbase = sid * ROWS_T
    nb = _nbatches(sid, NS)
    rows = (rows0, rows1)
    semg = (semg0, semg1)
    sems_ = (sems0, sems1)

    def run(p_hbm, s_hbm, r_hbm, m_hbm):
        pltpu.sync_copy(p_hbm.at[pl.ds(rbase, ROWS_T)],
                        acc.at[pl.ds(rbase, ROWS_T)])
        plsc.subcore_barrier()

        def off(i):
            return (sid + i * NS) * EB

        def load_idx(i, k):
            pltpu.async_copy(s_hbm.at[pl.ds(off(i), EB)], idxs.at[k], semi)
            pltpu.async_copy(r_hbm.at[pl.ds(off(i), EB)], idxr.at[k], semi)

        def wait_idx(k):
            pltpu.make_async_copy(s_hbm.at[pl.ds(0, EB)], idxs.at[k], semi).wait()
            pltpu.make_async_copy(r_hbm.at[pl.ds(0, EB)], idxr.at[k], semi).wait()

        def fire_gather(k):
            pltpu.async_copy(p_hbm.at[idxs.at[k]], rows[k], semg[k])

        def wait_gather(k):
            pltpu.make_async_copy(p_hbm.at[idxs.at[k]], rows[k], semg[k]).wait()

        def fire_scatter(k):
            pltpu.async_copy(rows[k], acc.at[idxr.at[k]], sems_[k], add=True)

        def wait_scatter(k):
            pltpu.make_async_copy(rows[k], acc.at[idxr.at[k]], sems_[k]).wait()

        load_idx(0, 0)
        wait_idx(0)
        fire_gather(0)

        def body(j, c):
            for k in range(2):
                i = 2 * j + k

                @pl.when(i < nb)
                def _():
                    @pl.when(i + 1 < nb)
                    def _():
                        load_idx(i + 1, 1 - k)

                    @pl.when(i >= 1)
                    def _():
                        wait_scatter(1 - k)

                    wait_gather(k)

                    @pl.when(i + 1 < nb)
                    def _():
                        wait_idx(1 - k)
                        fire_gather(1 - k)

                    fire_scatter(k)

            return c

        lax.fori_loop(0, (nb + 1) // 2, body, 0)

        @pl.when(nb % 2 == 1)
        def _():
            wait_scatter(0)

        @pl.when(nb % 2 == 0)
        def _():
            wait_scatter(1)

        plsc.subcore_barrier()
        pltpu.sync_copy(acc.at[pl.ds(rbase, ROWS_T)],
                        m_hbm.at[pl.ds(rbase, ROWS_T)])

    @pl.when(cid == 0)
    def _():
        run(p1, s1, r1, m1_o)

    @pl.when(cid == 1)
    def _():
        run(p2, s2, r2, m2_o)


_sc_conv = pl.kernel(_sc_conv_body, **_SC_CONV_KW)


_SC_EDGE_KW = dict(
    out_type=jax.ShapeDtypeStruct((E,), f32),
    mesh=_mesh,
    scratch_types=[
        pltpu.VMEM((EB,), i32),
        pltpu.VMEM((EB,), i32),
        pltpu.VMEM((EB, INNER), f32),
        pltpu.VMEM((EB, INNER), f32),
        pltpu.VMEM((EB * 16,), f32),
        pltpu.VMEM((EB,), f32),
        pltpu.SemaphoreType.DMA,
    ],
    compiler_params=pltpu.CompilerParams(needs_layout_passes=False),
)


def _sc_edge_body(l_h, s_h, r_h, out_o, idx_s, idx_r, ls, lr, tmp, ob, sem):
    cid = lax.axis_index("c")
    sid = lax.axis_index("s")
    wid = cid * NS + sid
    nb = _nbatches(wid, NC * NS)
    lane = lax.iota(i32, 16)

    def batch(i, c):
        off = (wid + i * NC * NS) * EB
        pltpu.sync_copy(s_h.at[pl.ds(off, EB)], idx_s)
        pltpu.sync_copy(r_h.at[pl.ds(off, EB)], idx_r)
        pltpu.async_copy(l_h.at[idx_s], ls, sem).wait()
        pltpu.async_copy(l_h.at[idx_r], lr, sem).wait()

        def edge(e, c2):
            vs = [ls[e, pl.ds(k * 16, 16)] * lr[e, pl.ds(k * 16, 16)]
                  for k in range(8)]
            while len(vs) > 1:
                vs = [vs[t] + vs[t + 1] for t in range(0, len(vs), 2)]
            tmp[pl.ds(e * 16, 16)] = vs[0]
            return c2

        lax.fori_loop(0, EB, edge, 0)

        def group(g, c2):
            vs = [plsc.load_gather(tmp, [lane * 16 + g * 256 + j])
                  for j in range(16)]
            while len(vs) > 1:
                vs = [vs[t] + vs[t + 1] for t in range(0, len(vs), 2)]
            ob[pl.ds(g * 16, 16)] = vs[0]
            return c2

        lax.fori_loop(0, EB // 16, group, 0)
        pltpu.sync_copy(ob, out_o.at[pl.ds(off, EB)])
        return c

    lax.fori_loop(0, nb, batch, 0)


_sc_edge = pl.kernel(_sc_edge_body, **_SC_EDGE_KW)


def _inv(deg_blk):
    return lax.rsqrt(deg_blk[:, 0:1])


def _tc_pre0_body(x, ds1, ds2, w1, b1, w2, b2, p1_o, p2_o):
    xv = x[...]
    p1_o[...] = (jnp.dot(xv, w1[...], preferred_element_type=f32) + b1[...]) * _inv(ds1[...])
    p2_o[...] = (jnp.dot(xv, w2[...], preferred_element_type=f32) + b2[...]) * _inv(ds2[...])


def _tc_pre0(x, ds1, ds2, w1, b1, w2, b2):
    blk = 1024
    row = pl.BlockSpec((blk, INNER), lambda i: (i, 0))
    deg = pl.BlockSpec((blk, INNER), lambda i: (i, 0))
    ws = pl.BlockSpec((INNER, INNER), lambda i: (0, 0))
    bs = pl.BlockSpec((1, INNER), lambda i: (0, 0))
    return pl.pallas_call(
        _tc_pre0_body,
        grid=(NF // blk,),
        in_specs=[row, deg, deg, ws, bs, ws, bs],
        out_specs=[row, row],
        out_shape=[jax.ShapeDtypeStruct((NF, INNER), f32)] * 2,
    )(x, ds1, ds2, w1, b1, w2, b2)


def _tc_mid_body(m1, m2, dr1, dr2, ds1, ds2, wma, wmb, bm_, w1, b1, w2, b2,
                 p1_o, p2_o):
    h1 = m1[...] * _inv(dr1[...])
    h2 = m2[...] * _inv(dr2[...])
    xv = jnp.dot(h1, wma[...], preferred_element_type=f32)
    xv = xv + jnp.dot(h2, wmb[...], preferred_element_type=f32)
    xv = jnp.maximum(xv + bm_[...], 0.0)
    p1_o[...] = (jnp.dot(xv, w1[...], preferred_element_type=f32) + b1[...]) * _inv(ds1[...])
    p2_o[...] = (jnp.dot(xv, w2[...], preferred_element_type=f32) + b2[...]) * _inv(ds2[...])


def _tc_mid(m1, m2, dr1, dr2, ds1, ds2, wma, wmb, bm_, w1, b1, w2, b2):
    blk = 1024
    row = pl.BlockSpec((blk, INNER), lambda i: (i, 0))
    deg = pl.BlockSpec((blk, INNER), lambda i: (i, 0))
    ws = pl.BlockSpec((INNER, INNER), lambda i: (0, 0))
    bs = pl.BlockSpec((1, INNER), lambda i: (0, 0))
    return pl.pallas_call(
        _tc_mid_body,
        grid=(NF // blk,),
        in_specs=[row, row, deg, deg, deg, deg, ws, ws, bs, ws, bs, ws, bs],
        out_specs=[row, row],
        out_shape=[jax.ShapeDtypeStruct((NF, INNER), f32)] * 2,
    )(m1, m2, dr1, dr2, ds1, ds2, wma, wmb, bm_, w1, b1, w2, b2)


def _tc_fin_body(m1, m2, dr1, dr2, wma, wmb, bm_, wlog, blog_, x_o, l_o):
    h1 = m1[...] * _inv(dr1[...])
    h2 = m2[...] * _inv(dr2[...])
    xv = jnp.dot(h1, wma[...], preferred_element_type=f32)
    xv = xv + jnp.dot(h2, wmb[...], preferred_element_type=f32)
    xv = jnp.maximum(xv + bm_[...], 0.0)
    x_o[...] = xv
    l_o[...] = jnp.dot(xv, wlog[...], preferred_element_type=f32) + blog_[...]


def _tc_fin(m1, m2, dr1, dr2, wma, wmb, bm_, wlog, blog_):
    blk = 1024
    row = pl.BlockSpec((blk, INNER), lambda i: (i, 0))
    deg = pl.BlockSpec((blk, INNER), lambda i: (i, 0))
    ws = pl.BlockSpec((INNER, INNER), lambda i: (0, 0))
    bs = pl.BlockSpec((1, INNER), lambda i: (0, 0))
    return pl.pallas_call(
        _tc_fin_body,
        grid=(NF // blk,),
        in_specs=[row, row, deg, deg, ws, ws, bs, ws, bs],
        out_specs=[row, row],
        out_shape=[jax.ShapeDtypeStruct((NF, INNER), f32)] * 2,
    )(m1, m2, dr1, dr2, wma, wmb, bm_, wlog, blog_)


def _tc_pool_body(x, v_o):
    xb = x[...]
    mask = lax.broadcasted_iota(i32, (1, P, INNER), 1) != 0
    xm = jnp.where(mask, xb, 0.0)
    v_o[...] = jnp.sum(xm, axis=1, keepdims=True)


def _tc_pool(xr):
    return pl.pallas_call(
        _tc_pool_body,
        grid=(P,),
        in_specs=[pl.BlockSpec((1, P, INNER), lambda i: (i, 0, 0))],
        out_specs=pl.BlockSpec((1, 1, INNER), lambda i: (i, 0, 0)),
        out_shape=jax.ShapeDtypeStruct((P, 1, INNER), f32),
    )(xr)


def _tc_head_body(vs, nn, wev, bev, wout, bout_, v_o):
    v = vs[...] * (1.0 / (nn[...] - 1.0))
    wv = wev[...]
    bv = bev[...]
    for l in range(N_EVAL):
        v = jnp.maximum(jnp.dot(v, wv[l], preferred_element_type=f32)
                        + bv[l:l + 1, :], 0.0)
    v_o[...] = jnp.tanh(jnp.dot(v, wout[...], preferred_element_type=f32)
                        + bout_[0, 0])


def _tc_head(vs, nn, wev, bev, wout_pad, bout_):
    full = lambda s: pl.BlockSpec(s, lambda: tuple(0 for _ in s))
    return pl.pallas_call(
        _tc_head_body,
        in_specs=[full((P, INNER)), full((P, 1)), full((N_EVAL, INNER, INNER)),
                  full((N_EVAL, INNER)), full((INNER, INNER)), full((1, 1))],
        out_specs=full((P, INNER)),
        out_shape=jax.ShapeDtypeStruct((P, INNER), f32),
    )(vs, nn, wev, bev, wout_pad, bout_)


def kernel(nodes, senders, receivers, grid_senders, grid_receivers, n_node,
           embed, Wg1, bg1, Wg2, bg2, Wm, bm, Wlog, blog, Weval, beval,
           Wout, bout):
    s1 = senders.astype(i32)
    r1 = receivers.astype(i32)
    s2 = grid_senders.astype(i32)
    r2 = grid_receivers.astype(i32)
    nodes_pad = jnp.zeros((NF,), i32).at[:N].set(nodes.astype(i32))

    x0 = _sc_embed(nodes_pad, embed)
    ones_nf = jnp.ones((NF, INNER), f32)
    dr1, dr2 = _sc_conv(ones_nf, ones_nf, s1, r1, s2, r2)
    ds1, ds2 = _sc_conv(ones_nf, ones_nf, r1, s1, r2, s2)

    p1, p2 = _tc_pre0(x0, ds1, ds2, Wg1[0], bg1[0][None], Wg2[0], bg2[0][None])
    for l in range(N_GNN):
        m1, m2 = _sc_conv(p1, p2, s1, r1, s2, r2)
        wma, wmb = Wm[l, :INNER], Wm[l, INNER:]
        if l < N_GNN - 1:
            p1, p2 = _tc_mid(m1, m2, dr1, dr2, ds1, ds2, wma, wmb,
                             bm[l][None], Wg1[l + 1], bg1[l + 1][None],
                             Wg2[l + 1], bg2[l + 1][None])
        else:
            x7, L = _tc_fin(m1, m2, dr1, dr2, wma, wmb, bm[l][None],
                            Wlog, blog[None])

    logits = _sc_edge(L, s1, r1)
    vsum = _tc_pool(x7[:N].reshape(P, N // P, INNER)).reshape(P, INNER)
    v = _tc_head(vsum, n_node.astype(f32).reshape(P, 1), Weval, beval,
                 jnp.pad(Wout, ((0, 0), (0, INNER - 1))), bout.reshape(1, 1))
    return logits, v[:, :1]


# speedup vs baseline: 10.7285x; 1.0954x over previous
import functools

import jax
import jax.numpy as jnp
from jax import lax
from jax.experimental import pallas as pl
from jax.experimental.pallas import tpu as pltpu
from jax.experimental.pallas import tpu_sc as plsc

N = 10000
E = 320000
P = 100
INNER = 128
N_GNN = 7
N_EVAL = 5

NC, NS = 2, 16
NF = 10240
ROWS_T = NF // NS
EB = 128
NB = E // EB

_mesh = plsc.VectorSubcoreMesh(core_axis_name="c", subcore_axis_name="s")
f32 = jnp.float32
i32 = jnp.int32


def _nbatches(tid, nworkers):
    rem = NB % nworkers
    return jnp.where(tid < rem, NB // nworkers + 1, NB // nworkers)


_SC_EMBED_KW = dict(
    out_type=jax.ShapeDtypeStruct((NF, INNER), f32),
    mesh=_mesh,
    scratch_types=[
        pltpu.VMEM((EB,), i32),
        pltpu.VMEM((EB, INNER), f32),
        pltpu.SemaphoreType.DMA,
    ],
)


def _sc_embed_body(nodes_h, embed_h, x_o, nidx, xbuf, sem):
    cid = lax.axis_index("c")
    sid = lax.axis_index("s")

    @pl.when(cid == 0)
    def _():
        def gstep(t, c):
            nb2 = sid * ROWS_T + t * EB
            pltpu.sync_copy(nodes_h.at[pl.ds(nb2, EB)], nidx)
            pltpu.async_copy(embed_h.at[nidx], xbuf, sem).wait()
            pltpu.sync_copy(xbuf, x_o.at[pl.ds(nb2, EB)])
            return c

        lax.fori_loop(0, ROWS_T // EB, gstep, 0)


_sc_embed = pl.kernel(_sc_embed_body, **_SC_EMBED_KW)


_SC_CONV_KW = dict(
    out_type=(
        jax.ShapeDtypeStruct((NF, INNER), f32),
        jax.ShapeDtypeStruct((NF, INNER), f32),
    ),
    mesh=_mesh,
    scratch_types=[
        pltpu.VMEM((2, EB), i32),
        pltpu.VMEM((2, EB), i32),
        pltpu.VMEM((EB, INNER), f32),
        pltpu.VMEM((EB, INNER), f32),
        pltpu.VMEM_SHARED((NF, INNER), f32),
        pltpu.SemaphoreType.DMA,
        pltpu.SemaphoreType.DMA,
        pltpu.SemaphoreType.DMA,
        pltpu.SemaphoreType.DMA,
        pltpu.SemaphoreType.DMA,
    ],
)


def _sc_conv_body(p1, p2, s1, r1, s2, r2, m1_o, m2_o,
                  idxs, idxr, rows0, rows1, acc,
                  semi, semg0, semg1, sems0, sems1):
    cid = lax.axis_index("c")
    sid = lax.axis_index("s")
    rbase = sid * ROWS_T
    nb = _nbatches(sid, NS)
    rows = (rows0, rows1)
    semg = (semg0, semg1)
    sems_ = (sems0, sems1)

    def run(p_hbm, s_hbm, r_hbm, m_hbm):
        pltpu.sync_copy(p_hbm.at[pl.ds(rbase, ROWS_T)],
                        acc.at[pl.ds(rbase, ROWS_T)])
        plsc.subcore_barrier()

        def off(i):
            return (sid + i * NS) * EB

        def load_idx(i, k):
            pltpu.async_copy(s_hbm.at[pl.ds(off(i), EB)], idxs.at[k], semi)
            pltpu.async_copy(r_hbm.at[pl.ds(off(i), EB)], idxr.at[k], semi)

        def wait_idx(k):
            pltpu.make_async_copy(s_hbm.at[pl.ds(0, EB)], idxs.at[k], semi).wait()
            pltpu.make_async_copy(r_hbm.at[pl.ds(0, EB)], idxr.at[k], semi).wait()

        def fire_gather(k):
            pltpu.async_copy(p_hbm.at[idxs.at[k]], rows[k], semg[k])

        def wait_gather(k):
            pltpu.make_async_copy(p_hbm.at[idxs.at[k]], rows[k], semg[k]).wait()

        def fire_scatter(k):
            pltpu.async_copy(rows[k], acc.at[idxr.at[k]], sems_[k], add=True)

        def wait_scatter(k):
            pltpu.make_async_copy(rows[k], acc.at[idxr.at[k]], sems_[k]).wait()

        load_idx(0, 0)
        wait_idx(0)
        fire_gather(0)

        def body(j, c):
            for k in range(2):
                i = 2 * j + k

                @pl.when(i < nb)
                def _():
                    @pl.when(i + 1 < nb)
                    def _():
                        load_idx(i + 1, 1 - k)

                    @pl.when(i >= 1)
                    def _():
                        wait_scatter(1 - k)

                    wait_gather(k)

                    @pl.when(i + 1 < nb)
                    def _():
                        wait_idx(1 - k)
                        fire_gather(1 - k)

                    fire_scatter(k)

            return c

        lax.fori_loop(0, (nb + 1) // 2, body, 0)

        @pl.when(nb % 2 == 1)
        def _():
            wait_scatter(0)

        @pl.when(nb % 2 == 0)
        def _():
            wait_scatter(1)

        plsc.subcore_barrier()
        pltpu.sync_copy(acc.at[pl.ds(rbase, ROWS_T)],
                        m_hbm.at[pl.ds(rbase, ROWS_T)])

    @pl.when(cid == 0)
    def _():
        run(p1, s1, r1, m1_o)

    @pl.when(cid == 1)
    def _():
        run(p2, s2, r2, m2_o)


_sc_conv = pl.kernel(_sc_conv_body, **_SC_CONV_KW)


_SC_EDGE_KW = dict(
    out_type=jax.ShapeDtypeStruct((E,), f32),
    mesh=_mesh,
    scratch_types=[
        pltpu.VMEM((2, EB), i32),
        pltpu.VMEM((2, EB), i32),
        pltpu.VMEM((EB, INNER), f32),
        pltpu.VMEM((EB, INNER), f32),
        pltpu.VMEM((EB, INNER), f32),
        pltpu.VMEM((EB, INNER), f32),
        pltpu.VMEM((EB * 16,), f32),
        pltpu.VMEM((EB,), f32),
        pltpu.SemaphoreType.DMA,
        pltpu.SemaphoreType.DMA,
        pltpu.SemaphoreType.DMA,
    ],
    compiler_params=pltpu.CompilerParams(needs_layout_passes=False),
)


def _sc_edge_body(l_h, s_h, r_h, out_o, idxs, idxr, ls0, lr0, ls1, lr1,
                  tmp, ob, semi, semg0, semg1):
    cid = lax.axis_index("c")
    sid = lax.axis_index("s")
    wid = cid * NS + sid
    nb = _nbatches(wid, NC * NS)
    lane = lax.iota(i32, 16)
    ls = (ls0, ls1)
    lr = (lr0, lr1)
    semg = (semg0, semg1)

    def off(i):
        return (wid + i * NC * NS) * EB

    def load_idx(i, k):
        pltpu.async_copy(s_h.at[pl.ds(off(i), EB)], idxs.at[k], semi)
        pltpu.async_copy(r_h.at[pl.ds(off(i), EB)], idxr.at[k], semi)

    def wait_idx(k):
        pltpu.make_async_copy(s_h.at[pl.ds(0, EB)], idxs.at[k], semi).wait()
        pltpu.make_async_copy(r_h.at[pl.ds(0, EB)], idxr.at[k], semi).wait()

    def fire_gathers(k):
        pltpu.async_copy(l_h.at[idxs.at[k]], ls[k], semg[k])
        pltpu.async_copy(l_h.at[idxr.at[k]], lr[k], semg[k])

    def wait_gathers(k):
        pltpu.make_async_copy(l_h.at[idxs.at[k]], ls[k], semg[k]).wait()
        pltpu.make_async_copy(l_h.at[idxr.at[k]], lr[k], semg[k]).wait()

    load_idx(0, 0)
    wait_idx(0)
    fire_gathers(0)

    def body(j, c):
        for k in range(2):
            i = 2 * j + k

            @pl.when(i < nb)
            def _():
                @pl.when(i + 1 < nb)
                def _():
                    load_idx(i + 1, 1 - k)

                wait_gathers(k)

                @pl.when(i + 1 < nb)
                def _():
                    wait_idx(1 - k)
                    fire_gathers(1 - k)

                def edge(e, c2):
                    vs = [ls[k][e, pl.ds(q * 16, 16)] * lr[k][e, pl.ds(q * 16, 16)]
                          for q in range(8)]
                    while len(vs) > 1:
                        vs = [vs[t] + vs[t + 1] for t in range(0, len(vs), 2)]
                    tmp[pl.ds(e * 16, 16)] = vs[0]
                    return c2

                lax.fori_loop(0, EB, edge, 0)

                def group(g, c2):
                    vs = [plsc.load_gather(tmp, [lane * 16 + g * 256 + j])
                          for j in range(16)]
                    while len(vs) > 1:
                        vs = [vs[t] + vs[t + 1] for t in range(0, len(vs), 2)]
                    ob[pl.ds(g * 16, 16)] = vs[0]
                    return c2

                lax.fori_loop(0, EB // 16, group, 0)
                pltpu.sync_copy(ob, out_o.at[pl.ds(off(i), EB)])

        return c

    lax.fori_loop(0, (nb + 1) // 2, body, 0)


_sc_edge = pl.kernel(_sc_edge_body, **_SC_EDGE_KW)


def _inv(deg_blk):
    return lax.rsqrt(deg_blk[:, 0:1])


def _tc_pre0_body(x, ds1, ds2, w1, b1, w2, b2, p1_o, p2_o):
    xv = x[...]
    p1_o[...] = (jnp.dot(xv, w1[...], preferred_element_type=f32) + b1[...]) * _inv(ds1[...])
    p2_o[...] = (jnp.dot(xv, w2[...], preferred_element_type=f32) + b2[...]) * _inv(ds2[...])


def _tc_pre0(x, ds1, ds2, w1, b1, w2, b2):
    blk = 1024
    row = pl.BlockSpec((blk, INNER), lambda i: (i, 0))
    deg = pl.BlockSpec((blk, INNER), lambda i: (i, 0))
    ws = pl.BlockSpec((INNER, INNER), lambda i: (0, 0))
    bs = pl.BlockSpec((1, INNER), lambda i: (0, 0))
    return pl.pallas_call(
        _tc_pre0_body,
        grid=(NF // blk,),
        in_specs=[row, deg, deg, ws, bs, ws, bs],
        out_specs=[row, row],
        out_shape=[jax.ShapeDtypeStruct((NF, INNER), f32)] * 2,
    )(x, ds1, ds2, w1, b1, w2, b2)


def _tc_mid_body(m1, m2, dr1, dr2, ds1, ds2, wma, wmb, bm_, w1, b1, w2, b2,
                 p1_o, p2_o):
    h1 = m1[...] * _inv(dr1[...])
    h2 = m2[...] * _inv(dr2[...])
    xv = jnp.dot(h1, wma[...], preferred_element_type=f32)
    xv = xv + jnp.dot(h2, wmb[...], preferred_element_type=f32)
    xv = jnp.maximum(xv + bm_[...], 0.0)
    p1_o[...] = (jnp.dot(xv, w1[...], preferred_element_type=f32) + b1[...]) * _inv(ds1[...])
    p2_o[...] = (jnp.dot(xv, w2[...], preferred_element_type=f32) + b2[...]) * _inv(ds2[...])


def _tc_mid(m1, m2, dr1, dr2, ds1, ds2, wma, wmb, bm_, w1, b1, w2, b2):
    blk = 1024
    row = pl.BlockSpec((blk, INNER), lambda i: (i, 0))
    deg = pl.BlockSpec((blk, INNER), lambda i: (i, 0))
    ws = pl.BlockSpec((INNER, INNER), lambda i: (0, 0))
    bs = pl.BlockSpec((1, INNER), lambda i: (0, 0))
    return pl.pallas_call(
        _tc_mid_body,
        grid=(NF // blk,),
        in_specs=[row, row, deg, deg, deg, deg, ws, ws, bs, ws, bs, ws, bs],
        out_specs=[row, row],
        out_shape=[jax.ShapeDtypeStruct((NF, INNER), f32)] * 2,
    )(m1, m2, dr1, dr2, ds1, ds2, wma, wmb, bm_, w1, b1, w2, b2)


def _tc_fin_body(m1, m2, dr1, dr2, wma, wmb, bm_, wlog, blog_, x_o, l_o):
    h1 = m1[...] * _inv(dr1[...])
    h2 = m2[...] * _inv(dr2[...])
    xv = jnp.dot(h1, wma[...], preferred_element_type=f32)
    xv = xv + jnp.dot(h2, wmb[...], preferred_element_type=f32)
    xv = jnp.maximum(xv + bm_[...], 0.0)
    x_o[...] = xv
    l_o[...] = jnp.dot(xv, wlog[...], preferred_element_type=f32) + blog_[...]


def _tc_fin(m1, m2, dr1, dr2, wma, wmb, bm_, wlog, blog_):
    blk = 1024
    row = pl.BlockSpec((blk, INNER), lambda i: (i, 0))
    deg = pl.BlockSpec((blk, INNER), lambda i: (i, 0))
    ws = pl.BlockSpec((INNER, INNER), lambda i: (0, 0))
    bs = pl.BlockSpec((1, INNER), lambda i: (0, 0))
    return pl.pallas_call(
        _tc_fin_body,
        grid=(NF // blk,),
        in_specs=[row, row, deg, deg, ws, ws, bs, ws, bs],
        out_specs=[row, row],
        out_shape=[jax.ShapeDtypeStruct((NF, INNER), f32)] * 2,
    )(m1, m2, dr1, dr2, wma, wmb, bm_, wlog, blog_)


def _tc_pool_body(x, v_o):
    xb = x[...]
    mask = lax.broadcasted_iota(i32, (1, P, INNER), 1) != 0
    xm = jnp.where(mask, xb, 0.0)
    v_o[...] = jnp.sum(xm, axis=1, keepdims=True)


def _tc_pool(xr):
    return pl.pallas_call(
        _tc_pool_body,
        grid=(P,),
        in_specs=[pl.BlockSpec((1, P, INNER), lambda i: (i, 0, 0))],
        out_specs=pl.BlockSpec((1, 1, INNER), lambda i: (i, 0, 0)),
        out_shape=jax.ShapeDtypeStruct((P, 1, INNER), f32),
    )(xr)


def _tc_head_body(vs, nn, wev, bev, wout, bout_, v_o):
    v = vs[...] * (1.0 / (nn[...] - 1.0))
    wv = wev[...]
    bv = bev[...]
    for l in range(N_EVAL):
        v = jnp.maximum(jnp.dot(v, wv[l], preferred_element_type=f32)
                        + bv[l:l + 1, :], 0.0)
    v_o[...] = jnp.tanh(jnp.dot(v, wout[...], preferred_element_type=f32)
                        + bout_[0, 0])


def _tc_head(vs, nn, wev, bev, wout_pad, bout_):
    full = lambda s: pl.BlockSpec(s, lambda: tuple(0 for _ in s))
    return pl.pallas_call(
        _tc_head_body,
        in_specs=[full((P, INNER)), full((P, 1)), full((N_EVAL, INNER, INNER)),
                  full((N_EVAL, INNER)), full((INNER, INNER)), full((1, 1))],
        out_specs=full((P, INNER)),
        out_shape=jax.ShapeDtypeStruct((P, INNER), f32),
    )(vs, nn, wev, bev, wout_pad, bout_)


def kernel(nodes, senders, receivers, grid_senders, grid_receivers, n_node,
           embed, Wg1, bg1, Wg2, bg2, Wm, bm, Wlog, blog, Weval, beval,
           Wout, bout):
    s1 = senders.astype(i32)
    r1 = receivers.astype(i32)
    s2 = grid_senders.astype(i32)
    r2 = grid_receivers.astype(i32)
    nodes_pad = jnp.zeros((NF,), i32).at[:N].set(nodes.astype(i32))

    x0 = _sc_embed(nodes_pad, embed)
    ones_nf = jnp.ones((NF, INNER), f32)
    dr1, dr2 = _sc_conv(ones_nf, ones_nf, s1, r1, s2, r2)
    ds1, ds2 = _sc_conv(ones_nf, ones_nf, r1, s1, r2, s2)

    p1, p2 = _tc_pre0(x0, ds1, ds2, Wg1[0], bg1[0][None], Wg2[0], bg2[0][None])
    for l in range(N_GNN):
        m1, m2 = _sc_conv(p1, p2, s1, r1, s2, r2)
        wma, wmb = Wm[l, :INNER], Wm[l, INNER:]
        if l < N_GNN - 1:
            p1, p2 = _tc_mid(m1, m2, dr1, dr2, ds1, ds2, wma, wmb,
                             bm[l][None], Wg1[l + 1], bg1[l + 1][None],
                             Wg2[l + 1], bg2[l + 1][None])
        else:
            x7, L = _tc_fin(m1, m2, dr1, dr2, wma, wmb, bm[l][None],
                            Wlog, blog[None])

    logits = _sc_edge(L, s1, r1)
    vsum = _tc_pool(x7[:N].reshape(P, N // P, INNER)).reshape(P, INNER)
    v = _tc_head(vsum, n_node.astype(f32).reshape(P, 1), Weval, beval,
                 jnp.pad(Wout, ((0, 0), (0, INNER - 1))), bout.reshape(1, 1))
    return logits, v[:, :1]


# speedup vs baseline: 12.6144x; 1.1758x over previous
import functools

import jax
import jax.numpy as jnp
from jax import lax
from jax.experimental import pallas as pl
from jax.experimental.pallas import tpu as pltpu
from jax.experimental.pallas import tpu_sc as plsc

N = 10000
E = 320000
P = 100
INNER = 128
N_GNN = 7
N_EVAL = 5

NC, NS = 2, 16
NF = 10240
ROWS_T = NF // NS
EB = 128
NB = E // EB

_mesh = plsc.VectorSubcoreMesh(core_axis_name="c", subcore_axis_name="s")
f32 = jnp.float32
i32 = jnp.int32


def _nbatches(tid, nworkers):
    rem = NB % nworkers
    return jnp.where(tid < rem, NB // nworkers + 1, NB // nworkers)


_SC_EMBED_KW = dict(
    out_type=jax.ShapeDtypeStruct((NF, INNER), f32),
    mesh=_mesh,
    scratch_types=[
        pltpu.VMEM((EB,), i32),
        pltpu.VMEM((EB, INNER), f32),
        pltpu.SemaphoreType.DMA,
    ],
)


def _sc_embed_body(nodes_h, embed_h, x_o, nidx, xbuf, sem):
    cid = lax.axis_index("c")
    sid = lax.axis_index("s")

    @pl.when(cid == 0)
    def _():
        def gstep(t, c):
            nb2 = sid * ROWS_T + t * EB
            pltpu.sync_copy(nodes_h.at[pl.ds(nb2, EB)], nidx)
            pltpu.async_copy(embed_h.at[nidx], xbuf, sem).wait()
            pltpu.sync_copy(xbuf, x_o.at[pl.ds(nb2, EB)])
            return c

        lax.fori_loop(0, ROWS_T // EB, gstep, 0)


_sc_embed = pl.kernel(_sc_embed_body, **_SC_EMBED_KW)


_SC_CONV_KW = dict(
    out_type=(
        jax.ShapeDtypeStruct((NF, INNER), f32),
        jax.ShapeDtypeStruct((NF, INNER), f32),
    ),
    mesh=_mesh,
    scratch_types=[
        pltpu.VMEM((2, EB), i32),
        pltpu.VMEM((2, EB), i32),
        pltpu.VMEM((EB, INNER), f32),
        pltpu.VMEM((EB, INNER), f32),
        pltpu.VMEM_SHARED((NF, INNER), f32),
        pltpu.SemaphoreType.DMA,
        pltpu.SemaphoreType.DMA,
        pltpu.SemaphoreType.DMA,
        pltpu.SemaphoreType.DMA,
        pltpu.SemaphoreType.DMA,
    ],
)


def _sc_conv_body(p1, p2, s1, r1, s2, r2, m1_o, m2_o,
                  idxs, idxr, rows0, rows1, acc,
                  semi, semg0, semg1, sems0, sems1):
    cid = lax.axis_index("c")
    sid = lax.axis_index("s")
    rbase = sid * ROWS_T
    nb = _nbatches(sid, NS)
    rows = (rows0, rows1)
    semg = (semg0, semg1)
    sems_ = (sems0, sems1)

    def run(p_hbm, s_hbm, r_hbm, m_hbm):
        pltpu.sync_copy(p_hbm.at[pl.ds(rbase, ROWS_T)],
                        acc.at[pl.ds(rbase, ROWS_T)])
        plsc.subcore_barrier()

        def off(i):
            return (sid + i * NS) * EB

        def load_idx(i, k):
            pltpu.async_copy(s_hbm.at[pl.ds(off(i), EB)], idxs.at[k], semi)
            pltpu.async_copy(r_hbm.at[pl.ds(off(i), EB)], idxr.at[k], semi)

        def wait_idx(k):
            pltpu.make_async_copy(s_hbm.at[pl.ds(0, EB)], idxs.at[k], semi).wait()
            pltpu.make_async_copy(r_hbm.at[pl.ds(0, EB)], idxr.at[k], semi).wait()

        def fire_gather(k):
            pltpu.async_copy(p_hbm.at[idxs.at[k]], rows[k], semg[k])

        def wait_gather(k):
            pltpu.make_async_copy(p_hbm.at[idxs.at[k]], rows[k], semg[k]).wait()

        def fire_scatter(k):
            pltpu.async_copy(rows[k], acc.at[idxr.at[k]], sems_[k], add=True)

        def wait_scatter(k):
            pltpu.make_async_copy(rows[k], acc.at[idxr.at[k]], sems_[k]).wait()

        load_idx(0, 0)
        wait_idx(0)
        fire_gather(0)

        def body(j, c):
            for k in range(2):
                i = 2 * j + k

                @pl.when(i < nb)
                def _():
                    @pl.when(i + 1 < nb)
                    def _():
                        load_idx(i + 1, 1 - k)

                    @pl.when(i >= 1)
                    def _():
                        wait_scatter(1 - k)

                    @pl.when(i + 1 < nb)
                    def _():
                        wait_idx(1 - k)
                        fire_gather(1 - k)

                    wait_gather(k)
                    fire_scatter(k)

            return c

        lax.fori_loop(0, (nb + 1) // 2, body, 0)

        @pl.when(nb % 2 == 1)
        def _():
            wait_scatter(0)

        @pl.when(nb % 2 == 0)
        def _():
            wait_scatter(1)

        plsc.subcore_barrier()
        pltpu.sync_copy(acc.at[pl.ds(rbase, ROWS_T)],
                        m_hbm.at[pl.ds(rbase, ROWS_T)])

    @pl.when(cid == 0)
    def _():
        run(p1, s1, r1, m1_o)

    @pl.when(cid == 1)
    def _():
        run(p2, s2, r2, m2_o)


_sc_conv = pl.kernel(_sc_conv_body, **_SC_CONV_KW)


_SC_EDGE_KW = dict(
    out_type=jax.ShapeDtypeStruct((E,), f32),
    mesh=_mesh,
    scratch_types=[
        pltpu.VMEM((2, EB), i32),
        pltpu.VMEM((2, EB), i32),
        pltpu.VMEM((EB, INNER), f32),
        pltpu.VMEM((EB, INNER), f32),
        pltpu.VMEM((EB, INNER), f32),
        pltpu.VMEM((EB, INNER), f32),
        pltpu.VMEM((EB * 16,), f32),
        pltpu.VMEM((EB,), f32),
        pltpu.SemaphoreType.DMA,
        pltpu.SemaphoreType.DMA,
        pltpu.SemaphoreType.DMA,
    ],
    compiler_params=pltpu.CompilerParams(needs_layout_passes=False),
)


def _sc_edge_body(l_h, s_h, r_h, out_o, idxs, idxr, ls0, lr0, ls1, lr1,
                  tmp, ob, semi, semg0, semg1):
    cid = lax.axis_index("c")
    sid = lax.axis_index("s")
    wid = cid * NS + sid
    nb = _nbatches(wid, NC * NS)
    lane = lax.iota(i32, 16)
    ls = (ls0, ls1)
    lr = (lr0, lr1)
    semg = (semg0, semg1)

    def off(i):
        return (wid + i * NC * NS) * EB

    def load_idx(i, k):
        pltpu.async_copy(s_h.at[pl.ds(off(i), EB)], idxs.at[k], semi)
        pltpu.async_copy(r_h.at[pl.ds(off(i), EB)], idxr.at[k], semi)

    def wait_idx(k):
        pltpu.make_async_copy(s_h.at[pl.ds(0, EB)], idxs.at[k], semi).wait()
        pltpu.make_async_copy(r_h.at[pl.ds(0, EB)], idxr.at[k], semi).wait()

    def fire_gathers(k):
        pltpu.async_copy(l_h.at[idxs.at[k]], ls[k], semg[k])
        pltpu.async_copy(l_h.at[idxr.at[k]], lr[k], semg[k])

    def wait_gathers(k):
        pltpu.make_async_copy(l_h.at[idxs.at[k]], ls[k], semg[k]).wait()
        pltpu.make_async_copy(l_h.at[idxr.at[k]], lr[k], semg[k]).wait()

    load_idx(0, 0)
    wait_idx(0)
    fire_gathers(0)

    def body(j, c):
        for k in range(2):
            i = 2 * j + k

            @pl.when(i < nb)
            def _():
                @pl.when(i + 1 < nb)
                def _():
                    load_idx(i + 1, 1 - k)

                wait_gathers(k)

                @pl.when(i + 1 < nb)
                def _():
                    wait_idx(1 - k)
                    fire_gathers(1 - k)

                def edge(e, c2):
                    vs = [ls[k][e, pl.ds(q * 16, 16)] * lr[k][e, pl.ds(q * 16, 16)]
                          for q in range(8)]
                    while len(vs) > 1:
                        vs = [vs[t] + vs[t + 1] for t in range(0, len(vs), 2)]
                    tmp[pl.ds(e * 16, 16)] = vs[0]
                    return c2

                lax.fori_loop(0, EB, edge, 0)

                def group(g, c2):
                    vs = [plsc.load_gather(tmp, [lane * 16 + g * 256 + j])
                          for j in range(16)]
                    while len(vs) > 1:
                        vs = [vs[t] + vs[t + 1] for t in range(0, len(vs), 2)]
                    ob[pl.ds(g * 16, 16)] = vs[0]
                    return c2

                lax.fori_loop(0, EB // 16, group, 0)
                pltpu.sync_copy(ob, out_o.at[pl.ds(off(i), EB)])

        return c

    lax.fori_loop(0, (nb + 1) // 2, body, 0)


_sc_edge = pl.kernel(_sc_edge_body, **_SC_EDGE_KW)


def _inv(deg_blk):
    return lax.rsqrt(deg_blk[:, 0:1])


def _tc_pre0_body(x, ds1, ds2, w1, b1, w2, b2, p1_o, p2_o):
    xv = x[...]
    p1_o[...] = (jnp.dot(xv, w1[...], preferred_element_type=f32) + b1[...]) * _inv(ds1[...])
    p2_o[...] = (jnp.dot(xv, w2[...], preferred_element_type=f32) + b2[...]) * _inv(ds2[...])


def _tc_pre0(x, ds1, ds2, w1, b1, w2, b2):
    blk = 1024
    row = pl.BlockSpec((blk, INNER), lambda i: (i, 0))
    deg = pl.BlockSpec((blk, INNER), lambda i: (i, 0))
    ws = pl.BlockSpec((INNER, INNER), lambda i: (0, 0))
    bs = pl.BlockSpec((1, INNER), lambda i: (0, 0))
    return pl.pallas_call(
        _tc_pre0_body,
        grid=(NF // blk,),
        in_specs=[row, deg, deg, ws, bs, ws, bs],
        out_specs=[row, row],
        out_shape=[jax.ShapeDtypeStruct((NF, INNER), f32)] * 2,
    )(x, ds1, ds2, w1, b1, w2, b2)


def _tc_mid_body(m1, m2, dr1, dr2, ds1, ds2, wma, wmb, bm_, w1, b1, w2, b2,
                 p1_o, p2_o):
    h1 = m1[...] * _inv(dr1[...])
    h2 = m2[...] * _inv(dr2[...])
    xv = jnp.dot(h1, wma[...], preferred_element_type=f32)
    xv = xv + jnp.dot(h2, wmb[...], preferred_element_type=f32)
    xv = jnp.maximum(xv + bm_[...], 0.0)
    p1_o[...] = (jnp.dot(xv, w1[...], preferred_element_type=f32) + b1[...]) * _inv(ds1[...])
    p2_o[...] = (jnp.dot(xv, w2[...], preferred_element_type=f32) + b2[...]) * _inv(ds2[...])


def _tc_mid(m1, m2, dr1, dr2, ds1, ds2, wma, wmb, bm_, w1, b1, w2, b2):
    blk = 1024
    row = pl.BlockSpec((blk, INNER), lambda i: (i, 0))
    deg = pl.BlockSpec((blk, INNER), lambda i: (i, 0))
    ws = pl.BlockSpec((INNER, INNER), lambda i: (0, 0))
    bs = pl.BlockSpec((1, INNER), lambda i: (0, 0))
    return pl.pallas_call(
        _tc_mid_body,
        grid=(NF // blk,),
        in_specs=[row, row, deg, deg, deg, deg, ws, ws, bs, ws, bs, ws, bs],
        out_specs=[row, row],
        out_shape=[jax.ShapeDtypeStruct((NF, INNER), f32)] * 2,
    )(m1, m2, dr1, dr2, ds1, ds2, wma, wmb, bm_, w1, b1, w2, b2)


def _tc_fin_body(m1, m2, dr1, dr2, wma, wmb, bm_, wlog, blog_, x_o, l_o):
    h1 = m1[...] * _inv(dr1[...])
    h2 = m2[...] * _inv(dr2[...])
    xv = jnp.dot(h1, wma[...], preferred_element_type=f32)
    xv = xv + jnp.dot(h2, wmb[...], preferred_element_type=f32)
    xv = jnp.maximum(xv + bm_[...], 0.0)
    x_o[...] = xv
    l_o[...] = jnp.dot(xv, wlog[...], preferred_element_type=f32) + blog_[...]


def _tc_fin(m1, m2, dr1, dr2, wma, wmb, bm_, wlog, blog_):
    blk = 1024
    row = pl.BlockSpec((blk, INNER), lambda i: (i, 0))
    deg = pl.BlockSpec((blk, INNER), lambda i: (i, 0))
    ws = pl.BlockSpec((INNER, INNER), lambda i: (0, 0))
    bs = pl.BlockSpec((1, INNER), lambda i: (0, 0))
    return pl.pallas_call(
        _tc_fin_body,
        grid=(NF // blk,),
        in_specs=[row, row, deg, deg, ws, ws, bs, ws, bs],
        out_specs=[row, row],
        out_shape=[jax.ShapeDtypeStruct((NF, INNER), f32)] * 2,
    )(m1, m2, dr1, dr2, wma, wmb, bm_, wlog, blog_)


def _tc_pool_body(x, v_o):
    xb = x[...]
    mask = lax.broadcasted_iota(i32, (1, P, INNER), 1) != 0
    xm = jnp.where(mask, xb, 0.0)
    v_o[...] = jnp.sum(xm, axis=1, keepdims=True)


def _tc_pool(xr):
    return pl.pallas_call(
        _tc_pool_body,
        grid=(P,),
        in_specs=[pl.BlockSpec((1, P, INNER), lambda i: (i, 0, 0))],
        out_specs=pl.BlockSpec((1, 1, INNER), lambda i: (i, 0, 0)),
        out_shape=jax.ShapeDtypeStruct((P, 1, INNER), f32),
    )(xr)


def _tc_head_body(vs, nn, wev, bev, wout, bout_, v_o):
    v = vs[...] * (1.0 / (nn[...] - 1.0))
    wv = wev[...]
    bv = bev[...]
    for l in range(N_EVAL):
        v = jnp.maximum(jnp.dot(v, wv[l], preferred_element_type=f32)
                        + bv[l:l + 1, :], 0.0)
    v_o[...] = jnp.tanh(jnp.dot(v, wout[...], preferred_element_type=f32)
                        + bout_[0, 0])


def _tc_head(vs, nn, wev, bev, wout_pad, bout_):
    full = lambda s: pl.BlockSpec(s, lambda: tuple(0 for _ in s))
    return pl.pallas_call(
        _tc_head_body,
        in_specs=[full((P, INNER)), full((P, 1)), full((N_EVAL, INNER, INNER)),
                  full((N_EVAL, INNER)), full((INNER, INNER)), full((1, 1))],
        out_specs=full((P, INNER)),
        out_shape=jax.ShapeDtypeStruct((P, INNER), f32),
    )(vs, nn, wev, bev, wout_pad, bout_)


def kernel(nodes, senders, receivers, grid_senders, grid_receivers, n_node,
           embed, Wg1, bg1, Wg2, bg2, Wm, bm, Wlog, blog, Weval, beval,
           Wout, bout):
    s1 = senders.astype(i32)
    r1 = receivers.astype(i32)
    s2 = grid_senders.astype(i32)
    r2 = grid_receivers.astype(i32)
    nodes_pad = jnp.zeros((NF,), i32).at[:N].set(nodes.astype(i32))

    x0 = _sc_embed(nodes_pad, embed)
    ones_nf = jnp.ones((NF, INNER), f32)
    dr1, dr2 = _sc_conv(ones_nf, ones_nf, s1, r1, s2, r2)
    ds1, ds2 = _sc_conv(ones_nf, ones_nf, r1, s1, r2, s2)

    p1, p2 = _tc_pre0(x0, ds1, ds2, Wg1[0], bg1[0][None], Wg2[0], bg2[0][None])
    for l in range(N_GNN):
        m1, m2 = _sc_conv(p1, p2, s1, r1, s2, r2)
        wma, wmb = Wm[l, :INNER], Wm[l, INNER:]
        if l < N_GNN - 1:
            p1, p2 = _tc_mid(m1, m2, dr1, dr2, ds1, ds2, wma, wmb,
                             bm[l][None], Wg1[l + 1], bg1[l + 1][None],
                             Wg2[l + 1], bg2[l + 1][None])
        else:
            x7, L = _tc_fin(m1, m2, dr1, dr2, wma, wmb, bm[l][None],
                            Wlog, blog[None])

    logits = _sc_edge(L, s1, r1)
    vsum = _tc_pool(x7[:N].reshape(P, N // P, INNER)).reshape(P, INNER)
    v = _tc_head(vsum, n_node.astype(f32).reshape(P, 1), Weval, beval,
                 jnp.pad(Wout, ((0, 0), (0, INNER - 1))), bout.reshape(1, 1))
    return logits, v[:, :1]


# speedup vs baseline: 13.5295x; 1.0725x over previous
import functools

import jax
import jax.numpy as jnp
from jax import lax
from jax.experimental import pallas as pl
from jax.experimental.pallas import tpu as pltpu
from jax.experimental.pallas import tpu_sc as plsc

N = 10000
E = 320000
P = 100
INNER = 128
N_GNN = 7
N_EVAL = 5

NC, NS = 2, 16
NF = 10240
ROWS_T = NF // NS
EB = 128
NB = E // EB

_mesh = plsc.VectorSubcoreMesh(core_axis_name="c", subcore_axis_name="s")
f32 = jnp.float32
i32 = jnp.int32


def _nbatches(tid, nworkers):
    rem = NB % nworkers
    return jnp.where(tid < rem, NB // nworkers + 1, NB // nworkers)


_SC_EMBED_KW = dict(
    out_type=jax.ShapeDtypeStruct((NF, INNER), f32),
    mesh=_mesh,
    scratch_types=[
        pltpu.VMEM((EB,), i32),
        pltpu.VMEM((EB, INNER), f32),
        pltpu.SemaphoreType.DMA,
    ],
)


def _sc_embed_body(nodes_h, embed_h, x_o, nidx, xbuf, sem):
    cid = lax.axis_index("c")
    sid = lax.axis_index("s")

    @pl.when(cid == 0)
    def _():
        def gstep(t, c):
            nb2 = sid * ROWS_T + t * EB
            pltpu.sync_copy(nodes_h.at[pl.ds(nb2, EB)], nidx)
            pltpu.async_copy(embed_h.at[nidx], xbuf, sem).wait()
            pltpu.sync_copy(xbuf, x_o.at[pl.ds(nb2, EB)])
            return c

        lax.fori_loop(0, ROWS_T // EB, gstep, 0)


_sc_embed = pl.kernel(_sc_embed_body, **_SC_EMBED_KW)


_SC_CONV_KW = dict(
    out_type=(
        jax.ShapeDtypeStruct((NF, INNER), f32),
        jax.ShapeDtypeStruct((NF, INNER), f32),
    ),
    mesh=_mesh,
    scratch_types=[
        pltpu.VMEM((2, EB), i32),
        pltpu.VMEM((2, EB), i32),
        pltpu.VMEM((EB, INNER), f32),
        pltpu.VMEM((EB, INNER), f32),
        pltpu.VMEM_SHARED((NF, INNER), f32),
        pltpu.SemaphoreType.DMA,
        pltpu.SemaphoreType.DMA,
        pltpu.SemaphoreType.DMA,
        pltpu.SemaphoreType.DMA,
        pltpu.SemaphoreType.DMA,
    ],
)


def _sc_conv_body(p1, p2, s1, r1, s2, r2, m1_o, m2_o,
                  idxs, idxr, rows0, rows1, acc,
                  semi, semg0, semg1, sems0, sems1):
    cid = lax.axis_index("c")
    sid = lax.axis_index("s")
    rbase = sid * ROWS_T
    nb = _nbatches(sid, NS)
    rows = (rows0, rows1)
    semg = (semg0, semg1)
    sems_ = (sems0, sems1)

    def run(p_hbm, s_hbm, r_hbm, m_hbm):
        pltpu.sync_copy(p_hbm.at[pl.ds(rbase, ROWS_T)],
                        acc.at[pl.ds(rbase, ROWS_T)])
        plsc.subcore_barrier()

        def off(i):
            return (sid + i * NS) * EB

        def load_idx(i, k):
            pltpu.async_copy(s_hbm.at[pl.ds(off(i), EB)], idxs.at[k], semi)
            pltpu.async_copy(r_hbm.at[pl.ds(off(i), EB)], idxr.at[k], semi)

        def wait_idx(k):
            pltpu.make_async_copy(s_hbm.at[pl.ds(0, EB)], idxs.at[k], semi).wait()
            pltpu.make_async_copy(r_hbm.at[pl.ds(0, EB)], idxr.at[k], semi).wait()

        def fire_gather(k):
            pltpu.async_copy(p_hbm.at[idxs.at[k]], rows[k], semg[k])

        def wait_gather(k):
            pltpu.make_async_copy(p_hbm.at[idxs.at[k]], rows[k], semg[k]).wait()

        def fire_scatter(k):
            pltpu.async_copy(rows[k], acc.at[idxr.at[k]], sems_[k], add=True)

        def wait_scatter(k):
            pltpu.make_async_copy(rows[k], acc.at[idxr.at[k]], sems_[k]).wait()

        load_idx(0, 0)
        wait_idx(0)
        fire_gather(0)

        def body(j, c):
            for k in range(2):
                i = 2 * j + k

                @pl.when(i < nb)
                def _():
                    @pl.when(i + 1 < nb)
                    def _():
                        load_idx(i + 1, 1 - k)

                    @pl.when(i >= 1)
                    def _():
                        wait_scatter(1 - k)

                    @pl.when(i + 1 < nb)
                    def _():
                        wait_idx(1 - k)
                        fire_gather(1 - k)

                    wait_gather(k)
                    fire_scatter(k)

            return c

        lax.fori_loop(0, (nb + 1) // 2, body, 0)

        @pl.when(nb % 2 == 1)
        def _():
            wait_scatter(0)

        @pl.when(nb % 2 == 0)
        def _():
            wait_scatter(1)

        plsc.subcore_barrier()
        pltpu.sync_copy(acc.at[pl.ds(rbase, ROWS_T)],
                        m_hbm.at[pl.ds(rbase, ROWS_T)])

    @pl.when(cid == 0)
    def _():
        run(p1, s1, r1, m1_o)

    @pl.when(cid == 1)
    def _():
        run(p2, s2, r2, m2_o)


_sc_conv = pl.kernel(_sc_conv_body, **_SC_CONV_KW)


_SC_DEG_KW = dict(
    out_type=tuple(jax.ShapeDtypeStruct((NF, INNER), f32) for _ in range(4)),
    mesh=_mesh,
    scratch_types=[
        pltpu.VMEM((2, EB), i32),
        pltpu.VMEM((EB, INNER), f32),
        pltpu.VMEM_SHARED((NF, INNER), f32),
        pltpu.SemaphoreType.DMA,
        pltpu.SemaphoreType.DMA,
        pltpu.SemaphoreType.DMA,
    ],
)


def _sc_deg_body(s1, r1, s2, r2, ones_h, ds1_o, dr1_o, ds2_o, dr2_o,
                 idx, ones_v, acc, semi, sems0, sems1):
    cid = lax.axis_index("c")
    sid = lax.axis_index("s")
    rbase = sid * ROWS_T
    nb = _nbatches(sid, NS)
    sems_ = (sems0, sems1)
    pltpu.sync_copy(ones_h, ones_v)

    def hist(e_hbm, out_hbm):
        for t in range(ROWS_T // EB):
            pltpu.sync_copy(ones_v, acc.at[pl.ds(rbase + t * EB, EB)])
        plsc.subcore_barrier()

        def load_idx(i, k):
            pltpu.async_copy(e_hbm.at[pl.ds((sid + i * NS) * EB, EB)],
                             idx.at[k], semi)

        def wait_idx(k):
            pltpu.make_async_copy(e_hbm.at[pl.ds(0, EB)], idx.at[k],
                                  semi).wait()

        def fire_scatter(k):
            pltpu.async_copy(ones_v, acc.at[idx.at[k]], sems_[k], add=True)

        def wait_scatter(k):
            pltpu.make_async_copy(ones_v, acc.at[idx.at[k]], sems_[k]).wait()

        load_idx(0, 0)

        def body(j, c):
            for k in range(2):
                i = 2 * j + k

                @pl.when(i < nb)
                def _():
                    @pl.when(i >= 1)
                    def _():
                        wait_scatter(1 - k)

                    @pl.when(i + 1 < nb)
                    def _():
                        load_idx(i + 1, 1 - k)

                    wait_idx(k)
                    fire_scatter(k)

            return c

        lax.fori_loop(0, (nb + 1) // 2, body, 0)

        @pl.when(nb % 2 == 1)
        def _():
            wait_scatter(0)

        @pl.when(nb % 2 == 0)
        def _():
            wait_scatter(1)

        plsc.subcore_barrier()
        pltpu.sync_copy(acc.at[pl.ds(rbase, ROWS_T)],
                        out_hbm.at[pl.ds(rbase, ROWS_T)])
        plsc.subcore_barrier()

    @pl.when(cid == 0)
    def _():
        hist(s1, ds1_o)
        hist(r1, dr1_o)

    @pl.when(cid == 1)
    def _():
        hist(s2, ds2_o)
        hist(r2, dr2_o)


_sc_deg = pl.kernel(_sc_deg_body, **_SC_DEG_KW)


_SC_EDGE_KW = dict(
    out_type=jax.ShapeDtypeStruct((E,), f32),
    mesh=_mesh,
    scratch_types=[
        pltpu.VMEM((2, EB), i32),
        pltpu.VMEM((2, EB), i32),
        pltpu.VMEM((EB, INNER), f32),
        pltpu.VMEM((EB, INNER), f32),
        pltpu.VMEM((EB, INNER), f32),
        pltpu.VMEM((EB, INNER), f32),
        pltpu.VMEM((EB * 16,), f32),
        pltpu.VMEM((EB,), f32),
        pltpu.SemaphoreType.DMA,
        pltpu.SemaphoreType.DMA,
        pltpu.SemaphoreType.DMA,
    ],
    compiler_params=pltpu.CompilerParams(needs_layout_passes=False),
)


def _sc_edge_body(l_h, s_h, r_h, out_o, idxs, idxr, ls0, lr0, ls1, lr1,
                  tmp, ob, semi, semg0, semg1):
    cid = lax.axis_index("c")
    sid = lax.axis_index("s")
    wid = cid * NS + sid
    nb = _nbatches(wid, NC * NS)
    lane = lax.iota(i32, 16)
    ls = (ls0, ls1)
    lr = (lr0, lr1)
    semg = (semg0, semg1)

    def off(i):
        return (wid + i * NC * NS) * EB

    def load_idx(i, k):
        pltpu.async_copy(s_h.at[pl.ds(off(i), EB)], idxs.at[k], semi)
        pltpu.async_copy(r_h.at[pl.ds(off(i), EB)], idxr.at[k], semi)

    def wait_idx(k):
        pltpu.make_async_copy(s_h.at[pl.ds(0, EB)], idxs.at[k], semi).wait()
        pltpu.make_async_copy(r_h.at[pl.ds(0, EB)], idxr.at[k], semi).wait()

    def fire_gathers(k):
        pltpu.async_copy(l_h.at[idxs.at[k]], ls[k], semg[k])
        pltpu.async_copy(l_h.at[idxr.at[k]], lr[k], semg[k])

    def wait_gathers(k):
        pltpu.make_async_copy(l_h.at[idxs.at[k]], ls[k], semg[k]).wait()
        pltpu.make_async_copy(l_h.at[idxr.at[k]], lr[k], semg[k]).wait()

    load_idx(0, 0)
    wait_idx(0)
    fire_gathers(0)

    def body(j, c):
        for k in range(2):
            i = 2 * j + k

            @pl.when(i < nb)
            def _():
                @pl.when(i + 1 < nb)
                def _():
                    load_idx(i + 1, 1 - k)

                wait_gathers(k)

                @pl.when(i + 1 < nb)
                def _():
                    wait_idx(1 - k)
                    fire_gathers(1 - k)

                def edge(e, c2):
                    vs = [ls[k][e, pl.ds(q * 16, 16)] * lr[k][e, pl.ds(q * 16, 16)]
                          for q in range(8)]
                    while len(vs) > 1:
                        vs = [vs[t] + vs[t + 1] for t in range(0, len(vs), 2)]
                    tmp[pl.ds(e * 16, 16)] = vs[0]
                    return c2

                lax.fori_loop(0, EB, edge, 0)

                def group(g, c2):
                    vs = [plsc.load_gather(tmp, [lane * 16 + g * 256 + j])
                          for j in range(16)]
                    while len(vs) > 1:
                        vs = [vs[t] + vs[t + 1] for t in range(0, len(vs), 2)]
                    ob[pl.ds(g * 16, 16)] = vs[0]
                    return c2

                lax.fori_loop(0, EB // 16, group, 0)
                pltpu.sync_copy(ob, out_o.at[pl.ds(off(i), EB)])

        return c

    lax.fori_loop(0, (nb + 1) // 2, body, 0)


_sc_edge = pl.kernel(_sc_edge_body, **_SC_EDGE_KW)


def _inv(deg_blk):
    return lax.rsqrt(deg_blk[:, 0:1])


def _tc_pre0_body(x, ds1, ds2, w1, b1, w2, b2, p1_o, p2_o):
    xv = x[...]
    p1_o[...] = (jnp.dot(xv, w1[...], preferred_element_type=f32) + b1[...]) * _inv(ds1[...])
    p2_o[...] = (jnp.dot(xv, w2[...], preferred_element_type=f32) + b2[...]) * _inv(ds2[...])


def _tc_pre0(x, ds1, ds2, w1, b1, w2, b2):
    blk = 1024
    row = pl.BlockSpec((blk, INNER), lambda i: (i, 0))
    deg = pl.BlockSpec((blk, INNER), lambda i: (i, 0))
    ws = pl.BlockSpec((INNER, INNER), lambda i: (0, 0))
    bs = pl.BlockSpec((1, INNER), lambda i: (0, 0))
    return pl.pallas_call(
        _tc_pre0_body,
        grid=(NF // blk,),
        in_specs=[row, deg, deg, ws, bs, ws, bs],
        out_specs=[row, row],
        out_shape=[jax.ShapeDtypeStruct((NF, INNER), f32)] * 2,
    )(x, ds1, ds2, w1, b1, w2, b2)


def _tc_mid_body(m1, m2, dr1, dr2, ds1, ds2, wma, wmb, bm_, w1, b1, w2, b2,
                 p1_o, p2_o):
    h1 = m1[...] * _inv(dr1[...])
    h2 = m2[...] * _inv(dr2[...])
    xv = jnp.dot(h1, wma[...], preferred_element_type=f32)
    xv = xv + jnp.dot(h2, wmb[...], preferred_element_type=f32)
    xv = jnp.maximum(xv + bm_[...], 0.0)
    p1_o[...] = (jnp.dot(xv, w1[...], preferred_element_type=f32) + b1[...]) * _inv(ds1[...])
    p2_o[...] = (jnp.dot(xv, w2[...], preferred_element_type=f32) + b2[...]) * _inv(ds2[...])


def _tc_mid(m1, m2, dr1, dr2, ds1, ds2, wma, wmb, bm_, w1, b1, w2, b2):
    blk = 1024
    row = pl.BlockSpec((blk, INNER), lambda i: (i, 0))
    deg = pl.BlockSpec((blk, INNER), lambda i: (i, 0))
    ws = pl.BlockSpec((INNER, INNER), lambda i: (0, 0))
    bs = pl.BlockSpec((1, INNER), lambda i: (0, 0))
    return pl.pallas_call(
        _tc_mid_body,
        grid=(NF // blk,),
        in_specs=[row, row, deg, deg, deg, deg, ws, ws, bs, ws, bs, ws, bs],
        out_specs=[row, row],
        out_shape=[jax.ShapeDtypeStruct((NF, INNER), f32)] * 2,
    )(m1, m2, dr1, dr2, ds1, ds2, wma, wmb, bm_, w1, b1, w2, b2)


def _tc_fin_body(m1, m2, dr1, dr2, wma, wmb, bm_, wlog, blog_, x_o, l_o):
    h1 = m1[...] * _inv(dr1[...])
    h2 = m2[...] * _inv(dr2[...])
    xv = jnp.dot(h1, wma[...], preferred_element_type=f32)
    xv = xv + jnp.dot(h2, wmb[...], preferred_element_type=f32)
    xv = jnp.maximum(xv + bm_[...], 0.0)
    x_o[...] = xv
    l_o[...] = jnp.dot(xv, wlog[...], preferred_element_type=f32) + blog_[...]


def _tc_fin(m1, m2, dr1, dr2, wma, wmb, bm_, wlog, blog_):
    blk = 1024
    row = pl.BlockSpec((blk, INNER), lambda i: (i, 0))
    deg = pl.BlockSpec((blk, INNER), lambda i: (i, 0))
    ws = pl.BlockSpec((INNER, INNER), lambda i: (0, 0))
    bs = pl.BlockSpec((1, INNER), lambda i: (0, 0))
    return pl.pallas_call(
        _tc_fin_body,
        grid=(NF // blk,),
        in_specs=[row, row, deg, deg, ws, ws, bs, ws, bs],
        out_specs=[row, row],
        out_shape=[jax.ShapeDtypeStruct((NF, INNER), f32)] * 2,
    )(m1, m2, dr1, dr2, wma, wmb, bm_, wlog, blog_)


def _tc_pool_body(x, v_o):
    xb = x[...]
    mask = lax.broadcasted_iota(i32, (1, P, INNER), 1) != 0
    xm = jnp.where(mask, xb, 0.0)
    v_o[...] = jnp.sum(xm, axis=1, keepdims=True)


def _tc_pool(xr):
    return pl.pallas_call(
        _tc_pool_body,
        grid=(P,),
        in_specs=[pl.BlockSpec((1, P, INNER), lambda i: (i, 0, 0))],
        out_specs=pl.BlockSpec((1, 1, INNER), lambda i: (i, 0, 0)),
        out_shape=jax.ShapeDtypeStruct((P, 1, INNER), f32),
    )(xr)


def _tc_head_body(vs, nn, wev, bev, wout, bout_, v_o):
    v = vs[...] * (1.0 / (nn[...] - 1.0))
    wv = wev[...]
    bv = bev[...]
    for l in range(N_EVAL):
        v = jnp.maximum(jnp.dot(v, wv[l], preferred_element_type=f32)
                        + bv[l:l + 1, :], 0.0)
    v_o[...] = jnp.tanh(jnp.dot(v, wout[...], preferred_element_type=f32)
                        + bout_[0, 0])


def _tc_head(vs, nn, wev, bev, wout_pad, bout_):
    full = lambda s: pl.BlockSpec(s, lambda: tuple(0 for _ in s))
    return pl.pallas_call(
        _tc_head_body,
        in_specs=[full((P, INNER)), full((P, 1)), full((N_EVAL, INNER, INNER)),
                  full((N_EVAL, INNER)), full((INNER, INNER)), full((1, 1))],
        out_specs=full((P, INNER)),
        out_shape=jax.ShapeDtypeStruct((P, INNER), f32),
    )(vs, nn, wev, bev, wout_pad, bout_)


def kernel(nodes, senders, receivers, grid_senders, grid_receivers, n_node,
           embed, Wg1, bg1, Wg2, bg2, Wm, bm, Wlog, blog, Weval, beval,
           Wout, bout):
    s1 = senders.astype(i32)
    r1 = receivers.astype(i32)
    s2 = grid_senders.astype(i32)
    r2 = grid_receivers.astype(i32)
    nodes_pad = jnp.zeros((NF,), i32).at[:N].set(nodes.astype(i32))

    x0 = _sc_embed(nodes_pad, embed)
    ones_eb = jnp.ones((EB, INNER), f32)
    ds1, dr1, ds2, dr2 = _sc_deg(s1, r1, s2, r2, ones_eb)

    p1, p2 = _tc_pre0(x0, ds1, ds2, Wg1[0], bg1[0][None], Wg2[0], bg2[0][None])
    for l in range(N_GNN):
        m1, m2 = _sc_conv(p1, p2, s1, r1, s2, r2)
        wma, wmb = Wm[l, :INNER], Wm[l, INNER:]
        if l < N_GNN - 1:
            p1, p2 = _tc_mid(m1, m2, dr1, dr2, ds1, ds2, wma, wmb,
                             bm[l][None], Wg1[l + 1], bg1[l + 1][None],
                             Wg2[l + 1], bg2[l + 1][None])
        else:
            x7, L = _tc_fin(m1, m2, dr1, dr2, wma, wmb, bm[l][None],
                            Wlog, blog[None])

    logits = _sc_edge(L, s1, r1)
    vsum = _tc_pool(x7[:N].reshape(P, N // P, INNER)).reshape(P, INNER)
    v = _tc_head(vsum, n_node.astype(f32).reshape(P, 1), Weval, beval,
                 jnp.pad(Wout, ((0, 0), (0, INNER - 1))), bout.reshape(1, 1))
    return logits, v[:, :1]
